# Initial kernel scaffold; baseline (speedup 1.0000x reference)
import jax
import jax.numpy as jnp
from jax.experimental import pallas as pl


def kernel(votes, vote_weights):
    raise NotImplementedError("write your pallas kernel here")


# speedup vs baseline: 5.0344x; 5.0344x over previous
import jax
import jax.numpy as jnp
from jax import lax
from jax.experimental import pallas as pl
from jax.experimental.pallas import tpu as pltpu
from jax.experimental.pallas import tpu_sc as plsc

NC = 2
NS = 16
L = 16


def _make_body(V, B, CB):
    NW = NC * NS
    BW = B // NW
    NCHUNK = BW // CB

    def body(votes_hbm, w_hbm, out_hbm, chunk_v, out_v, w_v):
        wid = lax.axis_index("s") * NC + lax.axis_index("c")
        base = wid * BW
        pltpu.sync_copy(w_hbm, w_v)
        wlo = w_v[pl.ds(0, L)]
        whi = w_v[pl.ds(L, L)]
        ws = [wlo[v] if v < L else whi[v - L] for v in range(V)]
        total = ws[0]
        for v in range(1, V):
            total = total + ws[v]

        def chunk_body(c, carry):
            col0 = base + c * CB
            pltpu.sync_copy(votes_hbm.at[:, pl.ds(col0, CB)], chunk_v)

            def group_body(g, carry2):
                sl = pl.ds(g * L, L)
                acc = ws[0] * chunk_v[0, sl].astype(jnp.float32)
                for v in range(1, V):
                    acc = acc + ws[v] * chunk_v[v, sl].astype(jnp.float32)
                out_v[sl] = jnp.where(acc + acc > total, 1, 0).astype(jnp.int32)
                return carry2

            lax.fori_loop(0, CB // L, group_body, 0)
            pltpu.sync_copy(out_v, out_hbm.at[pl.ds(col0, CB)])
            return carry

        lax.fori_loop(0, NCHUNK, chunk_body, 0)

    return body


def kernel(votes, vote_weights):
    V, B = votes.shape
    CB = 2048
    f = pl.kernel(
        _make_body(V, B, CB),
        out_type=jax.ShapeDtypeStruct((B,), jnp.int32),
        mesh=plsc.VectorSubcoreMesh(
            core_axis_name="c", subcore_axis_name="s",
            num_cores=NC, num_subcores=NS,
        ),
        scratch_types=[
            pltpu.VMEM((V, CB), jnp.int32),
            pltpu.VMEM((CB,), jnp.int32),
            pltpu.VMEM((2 * L,), jnp.float32),
        ],
    )
    w_pad = jnp.zeros((2 * L,), jnp.float32).at[:V].set(
        vote_weights.astype(jnp.float32))
    return f(votes, w_pad)
